# baseline (speedup 1.0000x reference)
import functools

import jax
import jax.numpy as jnp
from jax import lax
from jax.experimental import pallas as pl
from jax.experimental.pallas import tpu as pltpu
from jax.experimental.pallas import tpu_sc as plsc

T = 204800
VOCAB = 1000000
H = T // 2
D = 64
G = 96
RPG = 128
NC, NS = 2, 16
NW = NC * NS
TPW = T // NW
GPW = TPW // RPG
NBUF = 10
OUTER = GPW // NBUF

_sc_mesh = plsc.VectorSubcoreMesh(
    core_axis_name="c", subcore_axis_name="s", num_cores=NC, num_subcores=NS
)


@functools.partial(
    pl.kernel,
    out_type=jax.ShapeDtypeStruct((H, RPG), jnp.float32),
    mesh=_sc_mesh,
    compiler_params=pltpu.CompilerParams(use_tc_tiling_on_sc=False),
    scratch_types=[
        pltpu.VMEM((TPW,), jnp.int32),
        pltpu.VMEM((NBUF, RPG, D), jnp.float32),
        pltpu.SemaphoreType.DMA,
        pltpu.SemaphoreType.DMA,
    ],
)
def _sc_gather(idx_hbm, table_hbm, out_hbm, idx_v, rows_v, gsem, wsem):
    wid = lax.axis_index("s") * NC + lax.axis_index("c")
    half = wid // NS
    m0 = (wid % NS) * TPW
    col0 = half * D
    pltpu.sync_copy(idx_hbm.at[pl.ds(wid * TPW, TPW)], idx_v)

    def outer(o, _):
        jbase = o * NBUF
        gds = []
        for b in range(NBUF):
            ids = idx_v.at[pl.ds((jbase + b) * RPG, RPG)]
            gds.append(pltpu.async_copy(table_hbm.at[ids], rows_v.at[b], gsem))
        wds = []
        for b in range(NBUF):
            gds[b].wait()
            row0 = pl.multiple_of(m0 + (jbase + b) * RPG, RPG)
            wds.append(
                pltpu.async_copy(
                    rows_v.at[b], out_hbm.at[pl.ds(row0, RPG), pl.ds(col0, D)], wsem
                )
            )
        for b in range(NBUF):
            wds[b].wait()
        return _

    lax.fori_loop(0, OUTER, outer, None)


_TB = 2048
_NB = H // _TB


def _tc_body(g_ref, gz_ref, w_ref, b1_ref, b2_ref, o_ref):
    h = pl.program_id(1)
    blk = g_ref[...]
    g = jnp.where(h == 0, blk[:, :D], blk[:, D:])
    mm = lax.dot_general(
        w_ref[...], gz_ref[...], (((1,), (0,)), ((), ())),
        preferred_element_type=jnp.float32,
    )
    o_ref[...] = 2.0 * g.T + mm + b1_ref[...] + b2_ref[...]


_tc_fused = pl.pallas_call(
    _tc_body,
    grid=(_NB, 2),
    in_specs=[
        pl.BlockSpec((_TB, RPG), lambda i, h: (i, 0)),
        pl.BlockSpec((G, _TB), lambda i, h: (0, h * _NB + i)),
        pl.BlockSpec((D, G), lambda i, h: (0, 0)),
        pl.BlockSpec((D, 1), lambda i, h: (0, 0)),
        pl.BlockSpec((D, 1), lambda i, h: (0, 0)),
    ],
    out_specs=pl.BlockSpec((D, _TB), lambda i, h: (0, h * _NB + i)),
    out_shape=jax.ShapeDtypeStruct((D, T), jnp.float32),
)


def kernel(sentence_data, sentence_batch_sizes, gazetteers_data, word_table, W1, b1, W2, b2):
    gathered = _sc_gather(sentence_data, word_table)
    wc = jnp.concatenate([W1, W2], axis=1)
    gazT = gazetteers_data.T
    out_fm = _tc_fused(
        gathered, gazT, wc, b1.reshape(D, 1), b2.reshape(D, 1)
    )
    return (out_fm.T, sentence_batch_sizes)


# speedup vs baseline: 1.1515x; 1.1515x over previous
import functools

import jax
import jax.numpy as jnp
from jax import lax
from jax.experimental import pallas as pl
from jax.experimental.pallas import tpu as pltpu
from jax.experimental.pallas import tpu_sc as plsc

T = 204800
VOCAB = 1000000
D = 64
G = 96
RPG = 128
NC, NS = 2, 16
NW = NC * NS
TPW = T // NW
GPW = TPW // RPG
NBUF = 5
OUTER = GPW // NBUF

_sc_mesh = plsc.VectorSubcoreMesh(
    core_axis_name="c", subcore_axis_name="s", num_cores=NC, num_subcores=NS
)


@functools.partial(
    pl.kernel,
    out_type=jax.ShapeDtypeStruct((T, RPG), jnp.float32),
    mesh=_sc_mesh,
    compiler_params=pltpu.CompilerParams(use_tc_tiling_on_sc=False),
    scratch_types=[
        pltpu.VMEM((TPW,), jnp.int32),
        pltpu.VMEM((NBUF, RPG, RPG), jnp.float32),
        pltpu.SemaphoreType.DMA,
        pltpu.SemaphoreType.DMA,
    ],
)
def _sc_gather(idx_hbm, table_hbm, out_hbm, idx_v, rows_v, gsem, wsem):
    wid = lax.axis_index("s") * NC + lax.axis_index("c")
    m0 = wid * TPW
    pltpu.sync_copy(idx_hbm.at[pl.ds(m0, TPW)], idx_v)

    def outer(o, _):
        jbase = o * NBUF
        gds = []
        for b in range(NBUF):
            ids = idx_v.at[pl.ds((jbase + b) * RPG, RPG)]
            gds.append(pltpu.async_copy(table_hbm.at[ids], rows_v.at[b], gsem))
        wds = []
        for b in range(NBUF):
            gds[b].wait()
            row0 = pl.multiple_of(m0 + (jbase + b) * RPG, RPG)
            wds.append(
                pltpu.async_copy(rows_v.at[b], out_hbm.at[pl.ds(row0, RPG)], wsem)
            )
        for b in range(NBUF):
            wds[b].wait()
        return _

    lax.fori_loop(0, OUTER, outer, None)


_VC = 2048
_VGRID = -(-VOCAB // _VC)


def _pack_body(a_ref, o_ref):
    tr = a_ref[...].T
    o_ref[...] = jnp.concatenate([tr[: _VC // 2], tr[_VC // 2 :]], axis=1)


_tc_pack = pl.pallas_call(
    _pack_body,
    grid=(_VGRID,),
    in_specs=[pl.BlockSpec((D, _VC), lambda i: (0, i))],
    out_specs=pl.BlockSpec((_VC // 2, 2 * D), lambda i: (i, 0)),
    out_shape=jax.ShapeDtypeStruct((_VGRID * (_VC // 2), 2 * D), jnp.float32),
)


_TB = 2048
_NB = T // _TB


def _tc_body(g_ref, p_ref, gz_ref, w_ref, b1_ref, b2_ref, o_ref):
    blk = g_ref[...]
    g0 = blk[:, :D].T
    g1 = blk[:, D:].T
    p = p_ref[...]
    g = g0 + p * (g1 - g0)
    mm = lax.dot_general(
        w_ref[...], gz_ref[...], (((1,), (0,)), ((), ())),
        preferred_element_type=jnp.float32,
    )
    o_ref[...] = 2.0 * g + mm + b1_ref[...] + b2_ref[...]


_tc_fused = pl.pallas_call(
    _tc_body,
    grid=(_NB,),
    in_specs=[
        pl.BlockSpec((_TB, RPG), lambda i: (i, 0)),
        pl.BlockSpec((1, _TB), lambda i: (0, i)),
        pl.BlockSpec((G, _TB), lambda i: (0, i)),
        pl.BlockSpec((D, G), lambda i: (0, 0)),
        pl.BlockSpec((D, 1), lambda i: (0, 0)),
        pl.BlockSpec((D, 1), lambda i: (0, 0)),
    ],
    out_specs=pl.BlockSpec((D, _TB), lambda i: (0, i)),
    out_shape=jax.ShapeDtypeStruct((D, T), jnp.float32),
)


def kernel(sentence_data, sentence_batch_sizes, gazetteers_data, word_table, W1, b1, W2, b2):
    wt128 = _tc_pack(word_table.T)
    row_idx = ((sentence_data >> 11) << 10) | (sentence_data & 1023)
    parity = ((sentence_data >> 10) & 1).astype(jnp.float32).reshape(1, T)
    gathered = _sc_gather(row_idx, wt128)
    wc = jnp.concatenate([W1, W2], axis=1)
    gazT = gazetteers_data.T
    out_fm = _tc_fused(
        gathered, parity, gazT, wc, b1.reshape(D, 1), b2.reshape(D, 1)
    )
    return (out_fm.T, sentence_batch_sizes)


# speedup vs baseline: 1.2425x; 1.0791x over previous
import functools

import jax
import jax.numpy as jnp
from jax import lax
from jax.experimental import pallas as pl
from jax.experimental.pallas import tpu as pltpu
from jax.experimental.pallas import tpu_sc as plsc

T = 204800
H = T // 2
VOCAB = 1000000
D = 64
G = 96
RPG = 128
NC, NS = 2, 16
NW = NC * NS
TPW = T // NW
GPW = TPW // RPG
NBUF = 10
OUTER = GPW // NBUF

_VC = 2048
_VGRID = -(-VOCAB // _VC)
_NPACK = _VGRID * (_VC // 2)


def _pack_body(a_ref, o_ref):
    tr = a_ref[...].T
    o_ref[...] = jnp.concatenate([tr[: _VC // 2], tr[_VC // 2 :]], axis=1)


_tc_pack = pl.pallas_call(
    _pack_body,
    grid=(_VGRID,),
    in_specs=[pl.BlockSpec((D, _VC), lambda i: (0, i))],
    out_specs=pl.BlockSpec((_VC // 2, 2 * D), lambda i: (i, 0)),
    out_shape=jax.ShapeDtypeStruct((_NPACK, 2 * D), jnp.float32),
)


_sc_mesh = plsc.VectorSubcoreMesh(
    core_axis_name="c", subcore_axis_name="s", num_cores=NC, num_subcores=NS
)


@functools.partial(
    pl.kernel,
    out_type=jax.ShapeDtypeStruct((H, RPG), jnp.float32),
    mesh=_sc_mesh,
    compiler_params=pltpu.CompilerParams(use_tc_tiling_on_sc=False),
    scratch_types=[
        pltpu.VMEM((TPW,), jnp.int32),
        pltpu.VMEM((NBUF, RPG, D), jnp.float32),
        pltpu.SemaphoreType.DMA,
        pltpu.SemaphoreType.DMA,
    ],
)
def _sc_gather(idx_hbm, table_hbm, out_hbm, idx_v, rows_v, gsem, wsem):
    wid = lax.axis_index("s") * NC + lax.axis_index("c")
    half = wid // NS
    m0 = (wid % NS) * TPW
    col0 = half * D
    pltpu.sync_copy(idx_hbm.at[pl.ds(wid * TPW, TPW)], idx_v)

    def outer(o, _):
        jbase = o * NBUF
        gds = []
        for b in range(NBUF):
            ids = idx_v.at[pl.ds((jbase + b) * RPG, RPG)]
            gds.append(pltpu.async_copy(table_hbm.at[ids], rows_v.at[b], gsem))
        wds = []
        for b in range(NBUF):
            gds[b].wait()
            row0 = pl.multiple_of(m0 + (jbase + b) * RPG, RPG)
            wds.append(
                pltpu.async_copy(
                    rows_v.at[b], out_hbm.at[pl.ds(row0, RPG), pl.ds(col0, D)], wsem
                )
            )
        for b in range(NBUF):
            wds[b].wait()
        return _

    lax.fori_loop(0, OUTER, outer, None)


_TB = 2048
_NB = H // _TB


def _tc_body(g_ref, gz_ref, w_ref, b1_ref, b2_ref, o_ref):
    h = pl.program_id(1)
    blk = g_ref[...]
    g = jnp.where(h == 0, blk[:, :D], blk[:, D:])
    mm = lax.dot_general(
        w_ref[...], gz_ref[...], (((1,), (0,)), ((), ())),
        preferred_element_type=jnp.float32,
    )
    o_ref[...] = 2.0 * g.T + mm + b1_ref[...] + b2_ref[...]


_tc_fused = pl.pallas_call(
    _tc_body,
    grid=(_NB, 2),
    in_specs=[
        pl.BlockSpec((_TB, RPG), lambda i, h: (i, 0)),
        pl.BlockSpec((G, _TB), lambda i, h: (0, h * _NB + i)),
        pl.BlockSpec((D, G), lambda i, h: (0, 0)),
        pl.BlockSpec((D, 1), lambda i, h: (0, 0)),
        pl.BlockSpec((D, 1), lambda i, h: (0, 0)),
    ],
    out_specs=pl.BlockSpec((D, _TB), lambda i, h: (0, h * _NB + i)),
    out_shape=jax.ShapeDtypeStruct((D, T), jnp.float32),
)


def kernel(sentence_data, sentence_batch_sizes, gazetteers_data, word_table, W1, b1, W2, b2):
    wt128 = _tc_pack(word_table.T)
    wt64 = wt128.reshape(2 * _NPACK, D)
    packed_row = ((sentence_data >> 11) << 10) | (sentence_data & 1023)
    ids = (packed_row << 1) | ((sentence_data >> 10) & 1)
    gathered = _sc_gather(ids, wt64)
    wc = jnp.concatenate([W1, W2], axis=1)
    gazT = gazetteers_data.T
    out_fm = _tc_fused(
        gathered, gazT, wc, b1.reshape(D, 1), b2.reshape(D, 1)
    )
    return (out_fm.T, sentence_batch_sizes)
